# baseline (speedup 1.0000x reference)
import functools

import jax
import jax.numpy as jnp
from jax import lax
from jax.experimental import pallas as pl
from jax.experimental.pallas import tpu as pltpu
from jax.experimental.pallas import tpu_sc as plsc

NC = 2
NS = 16
NW = NC * NS
LANES = 16

DIM = 32
BATCH = 16384
TOT = 2 * BATCH
BPW = TOT // NW
CHUNK = 128
NCHUNK = BPW // CHUNK


def _tpu_kernel(e_weight, r_weight, h_idx, r_idx, t_idx):
    mesh = plsc.VectorSubcoreMesh(core_axis_name="c", subcore_axis_name="s")

    @functools.partial(
        pl.kernel,
        out_type=jax.ShapeDtypeStruct((TOT,), jnp.float32),
        mesh=mesh,
        compiler_params=pltpu.CompilerParams(
            needs_layout_passes=False, use_tc_tiling_on_sc=False),
        scratch_types=dict(
            hi=pltpu.VMEM((NCHUNK, CHUNK), jnp.int32),
            ri=pltpu.VMEM((NCHUNK, CHUNK), jnp.int32),
            ti=pltpu.VMEM((NCHUNK, CHUNK), jnp.int32),
            hrow=pltpu.VMEM((CHUNK, DIM), jnp.float32),
            rrow=pltpu.VMEM((CHUNK, DIM), jnp.float32),
            trow=pltpu.VMEM((CHUNK, DIM), jnp.float32),
            out_v=pltpu.VMEM((BPW,), jnp.float32),
            sem_h=pltpu.SemaphoreType.DMA,
            sem_r=pltpu.SemaphoreType.DMA,
            sem_t=pltpu.SemaphoreType.DMA,
        ),
    )
    def run(e_hbm, rel_hbm, hi_hbm, ri_hbm, ti_hbm, out_hbm,
            hi, ri, ti, hrow, rrow, trow, out_v, sem_h, sem_r, sem_t):
        wid = lax.axis_index("s") * NC + lax.axis_index("c")
        pltpu.sync_copy(hi_hbm.at[wid], hi)
        pltpu.sync_copy(ri_hbm.at[wid], ri)
        pltpu.sync_copy(ti_hbm.at[wid], ti)

        def chunk_body(c, carry):
            ch = pltpu.async_copy(e_hbm.at[hi.at[c]], hrow, sem_h)
            cr = pltpu.async_copy(rel_hbm.at[ri.at[c]], rrow, sem_r)
            ct = pltpu.async_copy(e_hbm.at[ti.at[c]], trow, sem_t)
            ch.wait()
            cr.wait()
            ct.wait()

            def grp(g, carry2):
                rows = lax.iota(jnp.int32, LANES) + g * LANES
                acc = jnp.zeros((LANES,), jnp.float32)
                for d in range(DIM):
                    cols = jnp.full((LANES,), d, jnp.int32)
                    hc = plsc.load_gather(hrow, [rows, cols])
                    rc = plsc.load_gather(rrow, [rows, cols])
                    tc = plsc.load_gather(trow, [rows, cols])
                    acc = acc + jnp.abs(hc + rc - tc)
                out_v[pl.ds(c * CHUNK + g * LANES, LANES)] = acc
                return carry2

            lax.fori_loop(0, CHUNK // LANES, grp, 0)
            return carry

        lax.fori_loop(0, NCHUNK, chunk_body, 0)
        pltpu.sync_copy(out_v, out_hbm.at[pl.ds(wid * BPW, BPW)])

    return run(e_weight, r_weight, h_idx, r_idx, t_idx)


def kernel(pos_triples, neg_triples, e_weight, r_weight):
    trip = jnp.concatenate(
        [pos_triples.astype(jnp.int32), neg_triples.astype(jnp.int32)], axis=1)
    h_idx = trip[0].reshape(NW, NCHUNK, CHUNK)
    r_idx = trip[1].reshape(NW, NCHUNK, CHUNK)
    t_idx = trip[2].reshape(NW, NCHUNK, CHUNK)
    out = _tpu_kernel(e_weight, r_weight, h_idx, r_idx, t_idx)
    return (out[:BATCH], out[BATCH:])
